# baseline (speedup 1.0000x reference)
import functools
import numpy as np
import jax
import jax.numpy as jnp
from jax import lax
from jax.experimental import pallas as pl
from jax.experimental.pallas import tpu as pltpu
from jax.experimental.pallas import tpu_sc as plsc


def _pts_table_t():
    vert_angles = np.radians(np.concatenate((
        np.linspace(4 + 1.0 / 3, -8 - 1.0 / 3, 40),
        np.linspace(-8 - 1.0 / 3 - 1.0 / 2, -24 - 1.0 / 3, 32))))
    hor_angles = np.radians(np.flip(np.arange(0, 360, 0.1728)) + 180)
    ray = np.array([1.0, 0, 0])
    vert_rotmat = np.array([[[np.cos(a), 0, -np.sin(a)], [0, 1, 0],
                             [np.sin(a), 0, np.cos(a)]] for a in vert_angles])
    hor_rotmat = np.array([[[np.cos(a), -np.sin(a), 0],
                            [np.sin(a), np.cos(a), 0],
                            [0, 0, 1]] for a in hor_angles])
    v = vert_rotmat @ ray
    pts = np.einsum('xij,yj->iyx', hor_rotmat, v)
    return pts.astype(np.float32)


_PTS_T = _pts_table_t()

_NB = 64
_PLANE = 72 * 2084
_NC = 2
_NS = 16
_NW = _NC * _NS
_STEP = 4688
_LEN = 4720


def _sc_xyz(data_hbm, pts_hbm, out_hbm,
            d0, m0, d1, m1,
            o00, o01, o02, o03, o10, o11, o12, o13,
            p0, p1, p2,
            s_in0, s_in1, s_out0, s_out1, s_pts):
    wid = lax.axis_index("s") * _NC + lax.axis_index("c")
    off = wid * _STEP

    din = ((d0, m0), (d1, m1))
    dout = ((o00, o01, o02, o03), (o10, o11, o12, o13))
    ptsb = (p0, p1, p2)

    for c in range(3):
        pltpu.async_copy(pts_hbm.at[pl.ds(c * _PLANE + off, _LEN)],
                         ptsb[c], s_pts)
    for c in range(3):
        pltpu.make_async_copy(pts_hbm.at[pl.ds(c * _PLANE + off, _LEN)],
                              ptsb[c], s_pts).wait()

    def issue_in(b, slot, sem):
        base = 2 * b * _PLANE + off
        pltpu.async_copy(data_hbm.at[pl.ds(base, _LEN)], din[slot][0], sem)
        pltpu.async_copy(data_hbm.at[pl.ds(base + _PLANE, _LEN)],
                         din[slot][1], sem)

    def wait_in(b, slot, sem):
        base = 2 * b * _PLANE + off
        pltpu.make_async_copy(data_hbm.at[pl.ds(base, _LEN)],
                              din[slot][0], sem).wait()
        pltpu.make_async_copy(data_hbm.at[pl.ds(base + _PLANE, _LEN)],
                              din[slot][1], sem).wait()

    def issue_out(b, slot, sem):
        for c in range(4):
            pltpu.async_copy(dout[slot][c],
                             out_hbm.at[pl.ds((4 * b + c) * _PLANE + off, _LEN)],
                             sem)

    def wait_out(b, slot, sem):
        for c in range(4):
            pltpu.make_async_copy(dout[slot][c],
                                  out_hbm.at[pl.ds((4 * b + c) * _PLANE + off, _LEN)],
                                  sem).wait()

    def compute(slot):
        dref, mref = din[slot]
        oref = dout[slot]

        def body(i, carry):
            sl = pl.ds(i * 16, 16)
            d = dref[sl]
            m = mref[sl]
            md = jnp.where(m >= 0.5, d, 0.0)
            oref[0][sl] = md * ptsb[0][sl]
            oref[1][sl] = md * ptsb[1][sl]
            oref[2][sl] = md * ptsb[2][sl]
            oref[3][sl] = m
            return carry
        lax.fori_loop(0, _LEN // 16, body, 0)

    sems_in = (s_in0, s_in1)
    sems_out = (s_out0, s_out1)

    issue_in(0, 0, s_in0)

    def outer(i, carry):
        for s in range(2):
            b = 2 * i + s
            wait_in(b, s, sems_in[s])

            @pl.when(b + 1 < _NB)
            def _():
                issue_in(b + 1, 1 - s, sems_in[1 - s])

            @pl.when(b >= 2)
            def _():
                wait_out(b - 2, s, sems_out[s])

            compute(s)
            issue_out(b, s, sems_out[s])
        return carry

    lax.fori_loop(0, _NB // 2, outer, 0)
    wait_out(_NB - 2, 0, s_out0)
    wait_out(_NB - 1, 1, s_out1)


_sc_call = functools.partial(
    pl.kernel,
    mesh=plsc.VectorSubcoreMesh(core_axis_name="c", subcore_axis_name="s"),
    out_type=jax.ShapeDtypeStruct((_NB * 4 * _PLANE,), jnp.float32),
    scratch_types=(
        [pltpu.VMEM((_LEN,), jnp.float32)] * 15
        + [pltpu.SemaphoreType.DMA] * 5
    ),
)(_sc_xyz)


def kernel(data):
    b, c, ys, xs = data.shape
    data1d = data.reshape(b * c * ys * xs)
    pts1d = _PTS_T[:, :ys, :xs].reshape(3 * ys * xs)
    out1d = _sc_call(data1d, pts1d)
    return out1d.reshape(b, 4, ys, xs)


# speedup vs baseline: 1.1771x; 1.1771x over previous
import functools
import numpy as np
import jax
import jax.numpy as jnp
from jax import lax
from jax.experimental import pallas as pl
from jax.experimental.pallas import tpu as pltpu
from jax.experimental.pallas import tpu_sc as plsc


def _pts_table_t():
    vert_angles = np.radians(np.concatenate((
        np.linspace(4 + 1.0 / 3, -8 - 1.0 / 3, 40),
        np.linspace(-8 - 1.0 / 3 - 1.0 / 2, -24 - 1.0 / 3, 32))))
    hor_angles = np.radians(np.flip(np.arange(0, 360, 0.1728)) + 180)
    ray = np.array([1.0, 0, 0])
    vert_rotmat = np.array([[[np.cos(a), 0, -np.sin(a)], [0, 1, 0],
                             [np.sin(a), 0, np.cos(a)]] for a in vert_angles])
    hor_rotmat = np.array([[[np.cos(a), -np.sin(a), 0],
                            [np.sin(a), np.cos(a), 0],
                            [0, 0, 1]] for a in hor_angles])
    v = vert_rotmat @ ray
    pts = np.einsum('xij,yj->iyx', hor_rotmat, v)
    return pts.astype(np.float32)


_PTS_T = _pts_table_t()

_NB = 64
_SPLIT = 56
_PLANE = 72 * 2084
_NC = 2
_NS = 16
_STEP = 4688
_LEN = 4720
_BB = 8


def _tc_xyz(data_ref, pts_ref, out_ref):
    for i in range(_BB):
        dist = data_ref[i, 0]
        maskv = data_ref[i, 1]
        m = maskv >= 0.5
        zero = jnp.zeros((), dtype=dist.dtype)
        md = jnp.where(m, dist, zero)
        out_ref[i, 0] = md * pts_ref[0]
        out_ref[i, 1] = md * pts_ref[1]
        out_ref[i, 2] = md * pts_ref[2]
        out_ref[i, 3] = maskv


def _tc_call(data, pts):
    b, c, ys, xs = data.shape
    return pl.pallas_call(
        _tc_xyz,
        grid=(_SPLIT // _BB,),
        in_specs=[
            pl.BlockSpec((_BB, c, ys, xs), lambda i: (i, 0, 0, 0)),
            pl.BlockSpec((3, ys, xs), lambda i: (0, 0, 0)),
        ],
        out_specs=pl.BlockSpec((_BB, 4, ys, xs), lambda i: (i, 0, 0, 0)),
        out_shape=jax.ShapeDtypeStruct((_SPLIT, 4, ys, xs), data.dtype),
        compiler_params=pltpu.CompilerParams(
            vmem_limit_bytes=100 * 1024 * 1024,
        ),
    )(data, pts)


_NB_SC = _NB - _SPLIT


def _sc_xyz(data_hbm, pts_hbm, out_hbm,
            d0, m0, d1, m1,
            o00, o01, o02, o03, o10, o11, o12, o13,
            p0, p1, p2,
            s_in0, s_in1, s_out0, s_out1, s_pts):
    wid = lax.axis_index("s") * _NC + lax.axis_index("c")
    off = wid * _STEP

    din = ((d0, m0), (d1, m1))
    dout = ((o00, o01, o02, o03), (o10, o11, o12, o13))
    ptsb = (p0, p1, p2)

    for c in range(3):
        pltpu.async_copy(pts_hbm.at[pl.ds(c * _PLANE + off, _LEN)],
                         ptsb[c], s_pts)
    for c in range(3):
        pltpu.make_async_copy(pts_hbm.at[pl.ds(c * _PLANE + off, _LEN)],
                              ptsb[c], s_pts).wait()

    def issue_in(b, slot, sem):
        base = 2 * (b + _SPLIT) * _PLANE + off
        pltpu.async_copy(data_hbm.at[pl.ds(base, _LEN)], din[slot][0], sem)
        pltpu.async_copy(data_hbm.at[pl.ds(base + _PLANE, _LEN)],
                         din[slot][1], sem)

    def wait_in(b, slot, sem):
        base = 2 * (b + _SPLIT) * _PLANE + off
        pltpu.make_async_copy(data_hbm.at[pl.ds(base, _LEN)],
                              din[slot][0], sem).wait()
        pltpu.make_async_copy(data_hbm.at[pl.ds(base + _PLANE, _LEN)],
                              din[slot][1], sem).wait()

    def issue_out(b, slot, sem):
        for c in range(4):
            pltpu.async_copy(dout[slot][c],
                             out_hbm.at[pl.ds((4 * b + c) * _PLANE + off, _LEN)],
                             sem)

    def wait_out(b, slot, sem):
        for c in range(4):
            pltpu.make_async_copy(dout[slot][c],
                                  out_hbm.at[pl.ds((4 * b + c) * _PLANE + off, _LEN)],
                                  sem).wait()

    def compute(slot):
        dref, mref = din[slot]
        oref = dout[slot]

        def body(i, carry):
            sl = pl.ds(i * 16, 16)
            d = dref[sl]
            m = mref[sl]
            md = jnp.where(m >= 0.5, d, 0.0)
            oref[0][sl] = md * ptsb[0][sl]
            oref[1][sl] = md * ptsb[1][sl]
            oref[2][sl] = md * ptsb[2][sl]
            oref[3][sl] = m
            return carry
        lax.fori_loop(0, _LEN // 16, body, 0)

    sems_in = (s_in0, s_in1)
    sems_out = (s_out0, s_out1)

    issue_in(0, 0, s_in0)

    def outer(i, carry):
        for s in range(2):
            b = 2 * i + s
            wait_in(b, s, sems_in[s])

            @pl.when(b + 1 < _NB_SC)
            def _():
                issue_in(b + 1, 1 - s, sems_in[1 - s])

            @pl.when(b >= 2)
            def _():
                wait_out(b - 2, s, sems_out[s])

            compute(s)
            issue_out(b, s, sems_out[s])
        return carry

    lax.fori_loop(0, _NB_SC // 2, outer, 0)
    wait_out(_NB_SC - 2, 0, s_out0)
    wait_out(_NB_SC - 1, 1, s_out1)


_sc_call = functools.partial(
    pl.kernel,
    mesh=plsc.VectorSubcoreMesh(core_axis_name="c", subcore_axis_name="s"),
    out_type=jax.ShapeDtypeStruct((_NB_SC * 4 * _PLANE,), jnp.float32),
    scratch_types=(
        [pltpu.VMEM((_LEN,), jnp.float32)] * 15
        + [pltpu.SemaphoreType.DMA] * 5
    ),
)(_sc_xyz)


def kernel(data):
    b, c, ys, xs = data.shape
    pts = _PTS_T[:, :ys, :xs]
    data1d = data.reshape(b * c * ys * xs)
    pts1d = pts.reshape(3 * ys * xs)
    sc_out = _sc_call(data1d, pts1d).reshape(_NB_SC, 4, ys, xs)
    tc_out = _tc_call(data, pts)
    return jnp.concatenate([tc_out, sc_out], axis=0)


# speedup vs baseline: 4.5147x; 3.8354x over previous
import numpy as np
import jax
import jax.numpy as jnp
from jax import lax
from jax.experimental import pallas as pl
from jax.experimental.pallas import tpu as pltpu


def _pts_table_t():
    vert_angles = np.radians(np.concatenate((
        np.linspace(4 + 1.0 / 3, -8 - 1.0 / 3, 40),
        np.linspace(-8 - 1.0 / 3 - 1.0 / 2, -24 - 1.0 / 3, 32))))
    hor_angles = np.radians(np.flip(np.arange(0, 360, 0.1728)) + 180)
    ray = np.array([1.0, 0, 0])
    vert_rotmat = np.array([[[np.cos(a), 0, -np.sin(a)], [0, 1, 0],
                             [np.sin(a), 0, np.cos(a)]] for a in vert_angles])
    hor_rotmat = np.array([[[np.cos(a), -np.sin(a), 0],
                            [np.sin(a), np.cos(a), 0],
                            [0, 0, 1]] for a in hor_angles])
    v = vert_rotmat @ ray
    pts = np.einsum('xij,yj->iyx', hor_rotmat, v)
    return pts.astype(np.float32)


_PTS_T = _pts_table_t()

_NB = 64
_CB = 2
_NCHUNK = _NB // _CB
_DEPTH = 4


def _xyz_kernel(data_hbm, pts_ref, out_hbm, inb, outb, *sems):
    sem_in = sems[:_DEPTH]
    sem_out = sems[_DEPTH:]

    def in_copy(chunk, slot, sem):
        return pltpu.make_async_copy(
            data_hbm.at[pl.ds(chunk * _CB, _CB)], inb.at[slot], sem)

    def out_copy(chunk, slot, sem):
        return pltpu.make_async_copy(
            outb.at[slot], out_hbm.at[pl.ds(chunk * _CB, _CB)], sem)

    for s in range(_DEPTH):
        in_copy(s, s, sem_in[s]).start()

    def outer(j, carry):
        for s in range(_DEPTH):
            chunk = j * _DEPTH + s
            in_copy(chunk, s, sem_in[s]).wait()

            @pl.when(chunk >= _DEPTH)
            def _():
                out_copy(chunk - _DEPTH, s, sem_out[s]).wait()

            for i in range(_CB):
                dist = inb[s, i, 0]
                maskv = inb[s, i, 1]
                md = jnp.where(maskv >= 0.5, dist,
                               jnp.zeros((), dtype=dist.dtype))
                outb[s, i, 0] = md * pts_ref[0]
                outb[s, i, 1] = md * pts_ref[1]
                outb[s, i, 2] = md * pts_ref[2]
                outb[s, i, 3] = maskv

            out_copy(chunk, s, sem_out[s]).start()

            @pl.when(chunk + _DEPTH < _NCHUNK)
            def _():
                in_copy(chunk + _DEPTH, s, sem_in[s]).start()
        return carry

    lax.fori_loop(0, _NCHUNK // _DEPTH, outer, 0)
    for s in range(_DEPTH):
        out_copy(_NCHUNK - _DEPTH + s, s, sem_out[s]).wait()


def kernel(data):
    b, c, ys, xs = data.shape
    pts = _PTS_T[:, :ys, :xs]
    return pl.pallas_call(
        _xyz_kernel,
        in_specs=[
            pl.BlockSpec(memory_space=pltpu.HBM),
            pl.BlockSpec(memory_space=pltpu.VMEM),
        ],
        out_specs=pl.BlockSpec(memory_space=pltpu.HBM),
        out_shape=jax.ShapeDtypeStruct((b, 4, ys, xs), data.dtype),
        scratch_shapes=(
            [pltpu.VMEM((_DEPTH, _CB, c, ys, xs), jnp.float32),
             pltpu.VMEM((_DEPTH, _CB, 4, ys, xs), jnp.float32)]
            + [pltpu.SemaphoreType.DMA] * (2 * _DEPTH)
        ),
        compiler_params=pltpu.CompilerParams(
            vmem_limit_bytes=100 * 1024 * 1024,
        ),
    )(data, pts)


# speedup vs baseline: 4.6685x; 1.0341x over previous
import numpy as np
import jax
import jax.numpy as jnp
from jax import lax
from jax.experimental import pallas as pl
from jax.experimental.pallas import tpu as pltpu


def _pts_table_t():
    vert_angles = np.radians(np.concatenate((
        np.linspace(4 + 1.0 / 3, -8 - 1.0 / 3, 40),
        np.linspace(-8 - 1.0 / 3 - 1.0 / 2, -24 - 1.0 / 3, 32))))
    hor_angles = np.radians(np.flip(np.arange(0, 360, 0.1728)) + 180)
    ray = np.array([1.0, 0, 0])
    vert_rotmat = np.array([[[np.cos(a), 0, -np.sin(a)], [0, 1, 0],
                             [np.sin(a), 0, np.cos(a)]] for a in vert_angles])
    hor_rotmat = np.array([[[np.cos(a), -np.sin(a), 0],
                            [np.sin(a), np.cos(a), 0],
                            [0, 0, 1]] for a in hor_angles])
    v = vert_rotmat @ ray
    pts = np.einsum('xij,yj->iyx', hor_rotmat, v)
    return pts.astype(np.float32)


_PTS_T = _pts_table_t()

_NB = 64
_CB = 4
_NCHUNK = _NB // _CB
_DEPTH = 4


def _xyz_kernel(data_hbm, pts_ref, out_hbm, inb, outb, *sems):
    sem_in = sems[:_DEPTH]
    sem_out = sems[_DEPTH:]

    def in_copy(chunk, slot, sem):
        return pltpu.make_async_copy(
            data_hbm.at[pl.ds(chunk * _CB, _CB)], inb.at[slot], sem)

    def out_copy(chunk, slot, sem):
        return pltpu.make_async_copy(
            outb.at[slot], out_hbm.at[pl.ds(chunk * _CB, _CB)], sem)

    for s in range(_DEPTH):
        in_copy(s, s, sem_in[s]).start()

    def outer(j, carry):
        for s in range(_DEPTH):
            chunk = j * _DEPTH + s
            in_copy(chunk, s, sem_in[s]).wait()

            @pl.when(chunk >= _DEPTH)
            def _():
                out_copy(chunk - _DEPTH, s, sem_out[s]).wait()

            for i in range(_CB):
                dist = inb[s, i, 0]
                maskv = inb[s, i, 1]
                md = jnp.where(maskv >= 0.5, dist,
                               jnp.zeros((), dtype=dist.dtype))
                outb[s, i, 0] = md * pts_ref[0]
                outb[s, i, 1] = md * pts_ref[1]
                outb[s, i, 2] = md * pts_ref[2]
                outb[s, i, 3] = maskv

            out_copy(chunk, s, sem_out[s]).start()

            @pl.when(chunk + _DEPTH < _NCHUNK)
            def _():
                in_copy(chunk + _DEPTH, s, sem_in[s]).start()
        return carry

    lax.fori_loop(0, _NCHUNK // _DEPTH, outer, 0)
    for s in range(_DEPTH):
        out_copy(_NCHUNK - _DEPTH + s, s, sem_out[s]).wait()


def kernel(data):
    b, c, ys, xs = data.shape
    pts = _PTS_T[:, :ys, :xs]
    return pl.pallas_call(
        _xyz_kernel,
        in_specs=[
            pl.BlockSpec(memory_space=pltpu.HBM),
            pl.BlockSpec(memory_space=pltpu.VMEM),
        ],
        out_specs=pl.BlockSpec(memory_space=pltpu.HBM),
        out_shape=jax.ShapeDtypeStruct((b, 4, ys, xs), data.dtype),
        scratch_shapes=(
            [pltpu.VMEM((_DEPTH, _CB, c, ys, xs), jnp.float32),
             pltpu.VMEM((_DEPTH, _CB, 4, ys, xs), jnp.float32)]
            + [pltpu.SemaphoreType.DMA] * (2 * _DEPTH)
        ),
        compiler_params=pltpu.CompilerParams(
            vmem_limit_bytes=100 * 1024 * 1024,
        ),
    )(data, pts)
